# baseline (speedup 1.0000x reference)
import functools

import jax
import jax.numpy as jnp
from jax import lax
from jax.experimental import pallas as pl
from jax.experimental.pallas import tpu as pltpu
from jax.experimental.pallas import tpu_sc as plsc

VOCAB_SIZE = 50257
DIM = 1024
CONTEXT_LENGTH = 2048
BATCH = 4
SEQ_LEN = 2048

_NC = 2
_NS = 16
_NW = _NC * _NS
_BS = BATCH * SEQ_LEN
_SW = SEQ_LEN // _NW
_C = 16
_SCHUNKS = _SW // _C
_NG = _SCHUNKS * BATCH
_NBUF = 5
_PBUF = 2
_LA = 3
_LANES = 16
_GROUPS = DIM // _LANES


def _make_kernel():
  mesh = plsc.VectorSubcoreMesh(core_axis_name="c", subcore_axis_name="s")

  @functools.partial(
      pl.kernel,
      out_type=jax.ShapeDtypeStruct((_BS, DIM), jnp.float32),
      mesh=mesh,
      scratch_types=[
          pltpu.VMEM((BATCH * _SW,), jnp.int32),
          pltpu.VMEM((_NBUF, _C, DIM), jnp.float32),
          pltpu.VMEM((_PBUF, _C, DIM), jnp.float32),
          pltpu.SemaphoreType.DMA((_NBUF,)),
          pltpu.SemaphoreType.DMA((_PBUF,)),
          pltpu.SemaphoreType.DMA((_NBUF,)),
      ],
  )
  def k(idx_hbm, tok_hbm, pos_hbm, out_hbm,
        idx_all, tok_v, pos_v, gsem, psem, osem):
    wid = lax.axis_index("s") * _NC + lax.axis_index("c")
    s0 = wid * _SW

    for b in range(BATCH):
      pltpu.sync_copy(idx_hbm.at[b, pl.ds(s0, _SW)],
                      idx_all.at[pl.ds(b * _SW, _SW)])

    def start_gather(c):
      sc, b = divmod(c, BATCH)
      pltpu.async_copy(
          tok_hbm.at[idx_all.at[pl.ds(b * _SW + sc * _C, _C)]],
          tok_v.at[c % _NBUF], gsem.at[c % _NBUF])

    def start_pos(sc):
      pltpu.async_copy(pos_hbm.at[pl.ds(s0 + sc * _C, _C)],
                       pos_v.at[sc % _PBUF], psem.at[sc % _PBUF])

    def wait_store(slot):
      pltpu.make_async_copy(
          tok_v.at[slot], out_hbm.at[pl.ds(0, _C)], osem.at[slot]).wait()

    start_pos(0)
    for c in range(_LA):
      start_gather(c)

    for c in range(_NG):
      sc, b = divmod(c, BATCH)
      u = c % _NBUF

      if c + _LA < _NG:
        if c + _LA >= _NBUF:
          wait_store((c + _LA) % _NBUF)
        start_gather(c + _LA)

      if b == 0:
        if sc + 1 < _SCHUNKS:
          start_pos(sc + 1)
        pltpu.make_async_copy(
            pos_hbm.at[pl.ds(0, _C)], pos_v.at[sc % _PBUF],
            psem.at[sc % _PBUF]).wait()

      pltpu.make_async_copy(
          tok_hbm.at[idx_all.at[pl.ds(0, _C)]], tok_v.at[u], gsem.at[u]
      ).wait()

      @pl.loop(0, _C)
      def add_row(i):
        for j in range(_GROUPS):
          sl = pl.ds(j * _LANES, _LANES)
          plsc.addupdate(tok_v.at[u, i, sl], pos_v[sc % _PBUF, i, sl])

      off = b * SEQ_LEN + s0 + sc * _C
      pltpu.async_copy(tok_v.at[u], out_hbm.at[pl.ds(off, _C)], osem.at[u])

    for t in range(_NBUF):
      wait_store(t)

  return k


_kernel_fn = _make_kernel()


def kernel(in_idx, tok_emb, pos_emb):
  out = _kernel_fn(in_idx.astype(jnp.int32), tok_emb, pos_emb)
  return out.reshape(BATCH, SEQ_LEN, DIM)
